# baseline (speedup 1.0000x reference)
import functools

import jax
import jax.numpy as jnp
from jax import lax
from jax.experimental import pallas as pl
from jax.experimental.pallas import tpu as pltpu
import jax.experimental.pallas.tpu_sc as plsc

N_NODES = 10000
E_EDGES = 320000
F = 128
EPS = 1e-5

NC = 2
NS = 16
N_PAD = 10240
ROWS_PER_TILE = N_PAD // NS
EDGES_PER_TILE = E_EDGES // (NC * NS)
E_HALF = E_EDGES // 2
EDGES_PER_TILE_H = E_HALF // (NC * NS)
CHUNK = 40
N_CHUNKS = EDGES_PER_TILE_H // CHUNK
WB = 40
CNT_W = 32

BLK_STATS = 2560
BLK_MM = 3200
BLK_CMB = 2000


def _prep_body(e_ref, g_ref, bt_ref, w_ref, b_ref, wp_ref, bp_ref, f_ref,
               s_acc, q_acc):
    p = pl.program_id(0)
    i = pl.program_id(1)

    @pl.when(p < 2)
    def _():
        blk = e_ref[...]
        s = jnp.sum(blk, axis=0, keepdims=True)
        q = jnp.sum(blk * blk, axis=0, keepdims=True)

        @pl.when(jnp.logical_and(p == 0, i == 0))
        def _():
            s_acc[...] = s
            q_acc[...] = q

        @pl.when(jnp.logical_or(p > 0, i > 0))
        def _():
            s_acc[...] += s
            q_acc[...] += q

        @pl.when(jnp.logical_and(p == 1, i == pl.num_programs(1) - 1))
        def _():
            inv_e = 1.0 / E_EDGES
            mean = s_acc[...] * inv_e
            var = q_acc[...] * inv_e - mean * mean
            scale = g_ref[...] * lax.rsqrt(var + EPS)
            shift = bt_ref[...] - mean * scale
            wp_ref[...] = w_ref[...] * jnp.transpose(scale)
            bp_ref[...] = (
                jnp.dot(shift, w_ref[...], preferred_element_type=jnp.float32,
                        precision=lax.Precision.HIGHEST)
                + b_ref[...]
            )

    @pl.when(p == 2)
    def _():
        x = e_ref[...]
        w = wp_ref[...]
        x_hi = x.astype(jnp.bfloat16)
        x_lo = (x - x_hi.astype(jnp.float32)).astype(jnp.bfloat16)
        w_hi = w.astype(jnp.bfloat16)
        w_lo = (w - w_hi.astype(jnp.float32)).astype(jnp.bfloat16)
        acc = jnp.dot(x_hi, w_lo, preferred_element_type=jnp.float32)
        acc = acc + jnp.dot(x_lo, w_hi, preferred_element_type=jnp.float32)
        acc = acc + jnp.dot(x_hi, w_hi, preferred_element_type=jnp.float32)
        f_ref[...] = jnp.maximum(acc + bp_ref[...], 0.0)


def _prep_call(e, g, bt, w, b):
    grid_i = E_HALF // BLK_MM
    return pl.pallas_call(
        _prep_body,
        grid=(3, grid_i),
        in_specs=[
            pl.BlockSpec((BLK_MM, F), lambda p, i: (i + (p % 2) * 50, 0)),
            pl.BlockSpec((1, F), lambda p, i: (0, 0)),
            pl.BlockSpec((1, F), lambda p, i: (0, 0)),
            pl.BlockSpec((F, F), lambda p, i: (0, 0)),
            pl.BlockSpec((1, F), lambda p, i: (0, 0)),
        ],
        out_specs=[
            pl.BlockSpec((F, F), lambda p, i: (0, 0)),
            pl.BlockSpec((1, F), lambda p, i: (0, 0)),
            pl.BlockSpec((BLK_MM, F), lambda p, i: ((p // 2) * i, 0)),
        ],
        out_shape=[
            jax.ShapeDtypeStruct((F, F), jnp.float32),
            jax.ShapeDtypeStruct((1, F), jnp.float32),
            jax.ShapeDtypeStruct((E_HALF, F), jnp.float32),
        ],
        scratch_shapes=[
            pltpu.VMEM((1, F), jnp.float32),
            pltpu.VMEM((1, F), jnp.float32),
        ],
    )(e, g, bt, w, b)


def _mm2_body(e_ref, wp_ref, bp_ref, f_ref):
    x = e_ref[...]
    w = wp_ref[...]
    x_hi = x.astype(jnp.bfloat16)
    x_lo = (x - x_hi.astype(jnp.float32)).astype(jnp.bfloat16)
    w_hi = w.astype(jnp.bfloat16)
    w_lo = (w - w_hi.astype(jnp.float32)).astype(jnp.bfloat16)
    acc = jnp.dot(x_hi, w_lo, preferred_element_type=jnp.float32)
    acc = acc + jnp.dot(x_lo, w_hi, preferred_element_type=jnp.float32)
    acc = acc + jnp.dot(x_hi, w_hi, preferred_element_type=jnp.float32)
    f_ref[...] = jnp.maximum(acc + bp_ref[...], 0.0)


def _mm2_call(e, wp, bp):
    grid_i = E_HALF // BLK_MM
    return pl.pallas_call(
        _mm2_body,
        grid=(grid_i,),
        in_specs=[
            pl.BlockSpec((BLK_MM, F), lambda i: (i + 50, 0)),
            pl.BlockSpec((F, F), lambda i: (0, 0)),
            pl.BlockSpec((1, F), lambda i: (0, 0)),
        ],
        out_specs=pl.BlockSpec((BLK_MM, F), lambda i: (i, 0)),
        out_shape=jax.ShapeDtypeStruct((E_HALF, F), jnp.float32),
    )(e, wp, bp)


def _sc_scatter_body(half, f_hbm, dst_hbm, zrow_hbm, ones_hbm, seq_hbm,
                     acc_hbm, cnt_hbm,
                     idx0, idx1, val0, val1, one_v, seq_v, acc_sh,
                     si0, si1, sv0, sv1):
    c = lax.axis_index("c")
    s = lax.axis_index("s")

    pltpu.sync_copy(zrow_hbm, val0)
    pltpu.sync_copy(ones_hbm, one_v)

    row0 = s * ROWS_PER_TILE
    out0 = c * N_PAD + row0
    base = (c * NS + s) * EDGES_PER_TILE_H
    dbase = half * E_HALF + base

    def zero_acc():
        for k in range(ROWS_PER_TILE // WB):
            pltpu.sync_copy(seq_hbm.at[pl.ds(row0 + k * WB, WB)], seq_v)
            pltpu.sync_copy(val0, acc_sh.at[seq_v])

    def read_acc(dst_ref):
        for k in range(ROWS_PER_TILE // WB):
            pltpu.sync_copy(seq_hbm.at[pl.ds(row0 + k * WB, WB)], seq_v)
            pltpu.sync_copy(acc_sh.at[seq_v], val0)
            pltpu.sync_copy(val0, dst_ref.at[pl.ds(out0 + k * WB, WB), :])

    def start_fill(chunk, idx_b, val_b, s_i, s_v):
        off = chunk * CHUNK
        pltpu.async_copy(dst_hbm.at[pl.ds(dbase + off, CHUNK)], idx_b, s_i)
        pltpu.async_copy(f_hbm.at[pl.ds(base + off, CHUNK), :], val_b, s_v)

    def wait_fill(chunk, idx_b, val_b, s_i, s_v):
        off = chunk * CHUNK
        pltpu.make_async_copy(dst_hbm.at[pl.ds(dbase + off, CHUNK)], idx_b,
                              s_i).wait()
        pltpu.make_async_copy(f_hbm.at[pl.ds(base + off, CHUNK), :], val_b,
                              s_v).wait()

    def start_idx(chunk, idx_b, s_i):
        off = dbase + chunk * CHUNK
        pltpu.async_copy(dst_hbm.at[pl.ds(off, CHUNK)], idx_b, s_i)

    def wait_idx(chunk, idx_b, s_i):
        off = dbase + chunk * CHUNK
        pltpu.make_async_copy(dst_hbm.at[pl.ds(off, CHUNK)], idx_b, s_i).wait()

    zero_acc()
    plsc.subcore_barrier()

    start_fill(0, idx0, val0, si0, sv0)

    def pair1(p, carry):
        c0 = 2 * p
        start_fill(c0 + 1, idx1, val1, si1, sv1)
        wait_fill(c0, idx0, val0, si0, sv0)
        pltpu.sync_copy(val0, acc_sh.at[idx0], add=True)
        start_fill(c0 + 2, idx0, val0, si0, sv0)
        wait_fill(c0 + 1, idx1, val1, si1, sv1)
        pltpu.sync_copy(val1, acc_sh.at[idx1], add=True)
        return carry

    lax.fori_loop(0, (N_CHUNKS - 1) // 2, pair1, 0)
    wait_fill(N_CHUNKS - 1, idx0, val0, si0, sv0)
    pltpu.sync_copy(val0, acc_sh.at[idx0], add=True)

    plsc.subcore_barrier()
    read_acc(acc_hbm)
    plsc.subcore_barrier()

    pltpu.sync_copy(zrow_hbm, val0)
    zero_acc()
    plsc.subcore_barrier()

    start_idx(0, idx0, si0)

    def pair2(p, carry):
        c0 = 2 * p
        start_idx(c0 + 1, idx1, si1)
        wait_idx(c0, idx0, si0)
        pltpu.sync_copy(one_v, acc_sh.at[idx0], add=True)
        start_idx(c0 + 2, idx0, si0)
        wait_idx(c0 + 1, idx1, si1)
        pltpu.sync_copy(one_v, acc_sh.at[idx1], add=True)
        return carry

    lax.fori_loop(0, (N_CHUNKS - 1) // 2, pair2, 0)
    wait_idx(N_CHUNKS - 1, idx0, si0)
    pltpu.sync_copy(one_v, acc_sh.at[idx0], add=True)

    plsc.subcore_barrier()
    read_acc(cnt_hbm)


@functools.cache
def _sc_scatter_fn(half):
    mesh = plsc.VectorSubcoreMesh(
        core_axis_name="c", subcore_axis_name="s", num_cores=NC, num_subcores=NS
    )
    return pl.kernel(
        functools.partial(_sc_scatter_body, half),
        out_type=[
            jax.ShapeDtypeStruct((NC * N_PAD, F), jnp.float32),
            jax.ShapeDtypeStruct((NC * N_PAD, F), jnp.float32),
        ],
        mesh=mesh,
        scratch_types=[
            pltpu.VMEM((CHUNK,), jnp.int32),
            pltpu.VMEM((CHUNK,), jnp.int32),
            pltpu.VMEM((CHUNK, F), jnp.float32),
            pltpu.VMEM((CHUNK, F), jnp.float32),
            pltpu.VMEM((CHUNK, F), jnp.float32),
            pltpu.VMEM((WB,), jnp.int32),
            pltpu.VMEM_SHARED((N_PAD, F), jnp.float32),
            pltpu.SemaphoreType.DMA,
            pltpu.SemaphoreType.DMA,
            pltpu.SemaphoreType.DMA,
            pltpu.SemaphoreType.DMA,
        ],
    )


def _combine_body(acc_a, cnt_a, acc_b, cnt_b, o_ref):
    a = acc_a[0] + acc_a[1] + acc_b[0] + acc_b[1]
    n = (cnt_a[0, :, 0:1] + cnt_a[1, :, 0:1]
         + cnt_b[0, :, 0:1] + cnt_b[1, :, 0:1])
    o_ref[...] = a / jnp.maximum(n, 1.0)


def _combine_call(acc, cnt, acc2, cnt2):
    grid = N_NODES // BLK_CMB
    return pl.pallas_call(
        _combine_body,
        grid=(grid,),
        in_specs=[
            pl.BlockSpec((NC, BLK_CMB, F), lambda i: (0, i, 0)),
            pl.BlockSpec((NC, BLK_CMB, F), lambda i: (0, i, 0)),
            pl.BlockSpec((NC, BLK_CMB, F), lambda i: (0, i, 0)),
            pl.BlockSpec((NC, BLK_CMB, F), lambda i: (0, i, 0)),
        ],
        out_specs=pl.BlockSpec((BLK_CMB, F), lambda i: (i, 0)),
        out_shape=jax.ShapeDtypeStruct((N_NODES, F), jnp.float32),
    )(acc, cnt, acc2, cnt2)


def kernel(e, edge_index, gamma, beta, W, b):
    dst = edge_index[1].astype(jnp.int32)
    g = gamma.reshape(1, F)
    bt = beta.reshape(1, F)
    b2 = b.reshape(1, F)

    wp, bp, f1 = _prep_call(e, g, bt, W, b2)
    f2 = _mm2_call(e, wp, bp)

    zrow = jnp.zeros((WB, F), jnp.float32)
    ones = jnp.ones((CHUNK, F), jnp.float32)
    seq = jnp.arange(N_PAD, dtype=jnp.int32)
    acc1, cnt1 = _sc_scatter_fn(0)(f1, dst, zrow, ones, seq)
    acc2, cnt2 = _sc_scatter_fn(1)(f2, dst, zrow, ones, seq)
    return _combine_call(acc1.reshape(NC, N_PAD, F),
                         cnt1.reshape(NC, N_PAD, F),
                         acc2.reshape(NC, N_PAD, F),
                         cnt2.reshape(NC, N_PAD, F))


# speedup vs baseline: 1.1752x; 1.1752x over previous
import functools

import jax
import jax.numpy as jnp
from jax import lax
from jax.experimental import pallas as pl
from jax.experimental.pallas import tpu as pltpu
import jax.experimental.pallas.tpu_sc as plsc

N_NODES = 10000
E_EDGES = 320000
F = 128
EPS = 1e-5

NC = 2
NS = 16
N_PAD = 10240
ROWS_PER_TILE = N_PAD // NS
EDGES_PER_TILE = E_EDGES // (NC * NS)
CHUNK = 80
N_CHUNKS = EDGES_PER_TILE // CHUNK
WB = 80
CNT_W = 32

BLK_STATS = 2560
BLK_MM = 2560
BLK_CMB = 2000


def _prep_body(e_ref, g_ref, bt_ref, w_ref, b_ref, wp_ref, bp_ref, f_ref,
               s_acc, q_acc):
    p = pl.program_id(0)
    i = pl.program_id(1)

    @pl.when(p == 0)
    def _():
        blk = e_ref[...]
        s = jnp.sum(blk, axis=0, keepdims=True)
        q = jnp.sum(blk * blk, axis=0, keepdims=True)

        @pl.when(i == 0)
        def _():
            s_acc[...] = s
            q_acc[...] = q

        @pl.when(i > 0)
        def _():
            s_acc[...] += s
            q_acc[...] += q

        @pl.when(i == pl.num_programs(1) - 1)
        def _():
            inv_e = 1.0 / E_EDGES
            mean = s_acc[...] * inv_e
            var = q_acc[...] * inv_e - mean * mean
            scale = g_ref[...] * lax.rsqrt(var + EPS)
            shift = bt_ref[...] - mean * scale
            wp_ref[...] = w_ref[...] * jnp.transpose(scale)
            bp_ref[...] = (
                jnp.dot(shift, w_ref[...], preferred_element_type=jnp.float32,
                        precision=lax.Precision.HIGHEST)
                + b_ref[...]
            )

    @pl.when(p == 1)
    def _():
        x = e_ref[...]
        w = wp_ref[...]
        x_hi = x.astype(jnp.bfloat16)
        x_lo = (x - x_hi.astype(jnp.float32)).astype(jnp.bfloat16)
        w_hi = w.astype(jnp.bfloat16)
        w_lo = (w - w_hi.astype(jnp.float32)).astype(jnp.bfloat16)
        acc = jnp.dot(x_hi, w_lo, preferred_element_type=jnp.float32)
        acc = acc + jnp.dot(x_lo, w_hi, preferred_element_type=jnp.float32)
        acc = acc + jnp.dot(x_hi, w_hi, preferred_element_type=jnp.float32)
        f_ref[...] = jnp.maximum(acc + bp_ref[...], 0.0)


def _prep_call(e, g, bt, w, b):
    grid_i = E_EDGES // BLK_MM
    return pl.pallas_call(
        _prep_body,
        grid=(2, grid_i),
        in_specs=[
            pl.BlockSpec((BLK_MM, F), lambda p, i: (i, 0)),
            pl.BlockSpec((1, F), lambda p, i: (0, 0)),
            pl.BlockSpec((1, F), lambda p, i: (0, 0)),
            pl.BlockSpec((F, F), lambda p, i: (0, 0)),
            pl.BlockSpec((1, F), lambda p, i: (0, 0)),
        ],
        out_specs=[
            pl.BlockSpec((F, F), lambda p, i: (0, 0)),
            pl.BlockSpec((1, F), lambda p, i: (0, 0)),
            pl.BlockSpec((BLK_MM, F), lambda p, i: (p * i, 0)),
        ],
        out_shape=[
            jax.ShapeDtypeStruct((F, F), jnp.float32),
            jax.ShapeDtypeStruct((1, F), jnp.float32),
            jax.ShapeDtypeStruct((E_EDGES, F), jnp.float32),
        ],
        scratch_shapes=[
            pltpu.VMEM((1, F), jnp.float32),
            pltpu.VMEM((1, F), jnp.float32),
        ],
    )(e, g, bt, w, b)


def _sc_scatter_body(f_hbm, dst_hbm, zrow_hbm, ones_hbm, seq_hbm,
                     acc_hbm, cnt_hbm,
                     idx0, idx1, val0, val1, one_v, seq_v, acc_sh,
                     si0, si1, sv0, sv1):
    c = lax.axis_index("c")
    s = lax.axis_index("s")
    nk = ROWS_PER_TILE // WB

    pltpu.sync_copy(zrow_hbm, val0)
    pltpu.sync_copy(ones_hbm, one_v)
    pltpu.sync_copy(seq_hbm.at[pl.ds(s * nk, nk), :], seq_v)

    row0 = s * ROWS_PER_TILE
    out0 = c * N_PAD + row0
    base = (c * NS + s) * EDGES_PER_TILE

    def zero_acc():
        for k in range(nk):
            pltpu.sync_copy(val0, acc_sh.at[seq_v.at[k]])

    def read_acc(dst_ref):
        bufs = (val0, val1)
        sems = (sv0, sv1)
        for k in range(nk):
            b = k % 2
            if k >= 2:
                pltpu.make_async_copy(
                    bufs[b],
                    dst_ref.at[pl.ds(out0 + (k - 2) * WB, WB), :],
                    sems[b]).wait()
            pltpu.sync_copy(acc_sh.at[seq_v.at[k]], bufs[b])
            pltpu.async_copy(bufs[b],
                             dst_ref.at[pl.ds(out0 + k * WB, WB), :], sems[b])
        for k in range(nk - 2, nk):
            b = k % 2
            pltpu.make_async_copy(
                bufs[b], dst_ref.at[pl.ds(out0 + k * WB, WB), :],
                sems[b]).wait()

    def start_fill(chunk, idx_b, val_b, s_i, s_v):
        off = base + chunk * CHUNK
        pltpu.async_copy(dst_hbm.at[pl.ds(off, CHUNK)], idx_b, s_i)
        pltpu.async_copy(f_hbm.at[pl.ds(off, CHUNK), :], val_b, s_v)

    def wait_fill(chunk, idx_b, val_b, s_i, s_v):
        off = base + chunk * CHUNK
        pltpu.make_async_copy(dst_hbm.at[pl.ds(off, CHUNK)], idx_b, s_i).wait()
        pltpu.make_async_copy(f_hbm.at[pl.ds(off, CHUNK), :], val_b, s_v).wait()

    def start_idx(chunk, idx_b, s_i):
        off = base + chunk * CHUNK
        pltpu.async_copy(dst_hbm.at[pl.ds(off, CHUNK)], idx_b, s_i)

    def wait_idx(chunk, idx_b, s_i):
        off = base + chunk * CHUNK
        pltpu.make_async_copy(dst_hbm.at[pl.ds(off, CHUNK)], idx_b, s_i).wait()

    zero_acc()
    plsc.subcore_barrier()

    start_fill(0, idx0, val0, si0, sv0)

    def pair1(p, carry):
        c0 = 2 * p
        start_fill(c0 + 1, idx1, val1, si1, sv1)
        wait_fill(c0, idx0, val0, si0, sv0)
        pltpu.sync_copy(val0, acc_sh.at[idx0], add=True)
        start_fill(c0 + 2, idx0, val0, si0, sv0)
        wait_fill(c0 + 1, idx1, val1, si1, sv1)
        pltpu.sync_copy(val1, acc_sh.at[idx1], add=True)
        return carry

    lax.fori_loop(0, (N_CHUNKS - 1) // 2, pair1, 0)
    wait_fill(N_CHUNKS - 1, idx0, val0, si0, sv0)
    pltpu.sync_copy(val0, acc_sh.at[idx0], add=True)

    plsc.subcore_barrier()
    read_acc(acc_hbm)
    plsc.subcore_barrier()

    pltpu.sync_copy(zrow_hbm, val0)
    pltpu.sync_copy(zrow_hbm, val1)
    zero_acc()
    plsc.subcore_barrier()

    start_idx(0, idx0, si0)

    def pair2(p, carry):
        c0 = 2 * p
        start_idx(c0 + 1, idx1, si1)
        wait_idx(c0, idx0, si0)
        pltpu.sync_copy(one_v, acc_sh.at[idx0], add=True)
        start_idx(c0 + 2, idx0, si0)
        wait_idx(c0 + 1, idx1, si1)
        pltpu.sync_copy(one_v, acc_sh.at[idx1], add=True)
        return carry

    lax.fori_loop(0, (N_CHUNKS - 1) // 2, pair2, 0)
    wait_idx(N_CHUNKS - 1, idx0, si0)
    pltpu.sync_copy(one_v, acc_sh.at[idx0], add=True)

    plsc.subcore_barrier()
    read_acc(cnt_hbm)


@functools.cache
def _sc_scatter_fn():
    mesh = plsc.VectorSubcoreMesh(
        core_axis_name="c", subcore_axis_name="s", num_cores=NC, num_subcores=NS
    )
    return pl.kernel(
        _sc_scatter_body,
        out_type=[
            jax.ShapeDtypeStruct((NC * N_PAD, F), jnp.float32),
            jax.ShapeDtypeStruct((NC * N_PAD, F), jnp.float32),
        ],
        mesh=mesh,
        scratch_types=[
            pltpu.VMEM((CHUNK,), jnp.int32),
            pltpu.VMEM((CHUNK,), jnp.int32),
            pltpu.VMEM((CHUNK, F), jnp.float32),
            pltpu.VMEM((CHUNK, F), jnp.float32),
            pltpu.VMEM((CHUNK, F), jnp.float32),
            pltpu.VMEM((ROWS_PER_TILE // WB, WB), jnp.int32),
            pltpu.VMEM_SHARED((N_PAD, F), jnp.float32),
            pltpu.SemaphoreType.DMA,
            pltpu.SemaphoreType.DMA,
            pltpu.SemaphoreType.DMA,
            pltpu.SemaphoreType.DMA,
        ],
    )


def _combine_body(acc_ref, cnt_ref, o_ref):
    a = acc_ref[0] + acc_ref[1]
    n = cnt_ref[0, :, 0:1] + cnt_ref[1, :, 0:1]
    o_ref[...] = a / jnp.maximum(n, 1.0)


def _combine_call(acc, cnt):
    grid = N_NODES // BLK_CMB
    return pl.pallas_call(
        _combine_body,
        grid=(grid,),
        in_specs=[
            pl.BlockSpec((NC, BLK_CMB, F), lambda i: (0, i, 0)),
            pl.BlockSpec((NC, BLK_CMB, F), lambda i: (0, i, 0)),
        ],
        out_specs=pl.BlockSpec((BLK_CMB, F), lambda i: (i, 0)),
        out_shape=jax.ShapeDtypeStruct((N_NODES, F), jnp.float32),
    )(acc, cnt)


def kernel(e, edge_index, gamma, beta, W, b):
    dst = edge_index[1].astype(jnp.int32)
    g = gamma.reshape(1, F)
    bt = beta.reshape(1, F)
    b2 = b.reshape(1, F)

    _, _, f = _prep_call(e, g, bt, W, b2)

    zrow = jnp.zeros((WB, F), jnp.float32)
    ones = jnp.ones((CHUNK, F), jnp.float32)
    seq = jnp.arange(N_PAD, dtype=jnp.int32).reshape(-1, WB)
    acc, cnt = _sc_scatter_fn()(f, dst, zrow, ones, seq)
    return _combine_call(acc.reshape(NC, N_PAD, F),
                         cnt.reshape(NC, N_PAD, F))


# speedup vs baseline: 1.1821x; 1.0059x over previous
import functools

import jax
import jax.numpy as jnp
from jax import lax
from jax.experimental import pallas as pl
from jax.experimental.pallas import tpu as pltpu
import jax.experimental.pallas.tpu_sc as plsc

N_NODES = 10000
E_EDGES = 320000
F = 128
EPS = 1e-5

NC = 2
NS = 16
N_PAD = 10240
ROWS_PER_TILE = N_PAD // NS
EDGES_PER_TILE = E_EDGES // (NC * NS)
CHUNK = 80
N_CHUNKS = EDGES_PER_TILE // CHUNK
WB = 80
CNT_W = 32

BLK_STATS = 2560
BLK_MM = 2560
BLK_CMB = 2000


def _prep_body(e_ref, g_ref, bt_ref, w_ref, b_ref, wp_ref, bp_ref, f_ref,
               s_acc, q_acc):
    p = pl.program_id(0)
    i = pl.program_id(1)

    @pl.when(p == 0)
    def _():
        blk = e_ref[...]
        s = jnp.sum(blk, axis=0, keepdims=True)
        q = jnp.sum(blk * blk, axis=0, keepdims=True)

        @pl.when(i == 0)
        def _():
            s_acc[...] = s
            q_acc[...] = q

        @pl.when(i > 0)
        def _():
            s_acc[...] += s
            q_acc[...] += q

        @pl.when(i == pl.num_programs(1) - 1)
        def _():
            inv_e = 1.0 / E_EDGES
            mean = s_acc[...] * inv_e
            var = q_acc[...] * inv_e - mean * mean
            scale = g_ref[...] * lax.rsqrt(var + EPS)
            shift = bt_ref[...] - mean * scale
            wp_ref[...] = w_ref[...] * jnp.transpose(scale)
            bp_ref[...] = (
                jnp.dot(shift, w_ref[...], preferred_element_type=jnp.float32,
                        precision=lax.Precision.HIGHEST)
                + b_ref[...]
            )

    @pl.when(p == 1)
    def _():
        x = e_ref[...]
        w = wp_ref[...]
        x_hi = x.astype(jnp.bfloat16)
        x_lo = (x - x_hi.astype(jnp.float32)).astype(jnp.bfloat16)
        w_hi = w.astype(jnp.bfloat16)
        w_lo = (w - w_hi.astype(jnp.float32)).astype(jnp.bfloat16)
        acc = jnp.dot(x_hi, w_lo, preferred_element_type=jnp.float32)
        acc = acc + jnp.dot(x_lo, w_hi, preferred_element_type=jnp.float32)
        acc = acc + jnp.dot(x_hi, w_hi, preferred_element_type=jnp.float32)
        f_ref[...] = jnp.maximum(acc + bp_ref[...], 0.0)


def _prep_call(e, g, bt, w, b):
    grid_i = E_EDGES // BLK_MM
    return pl.pallas_call(
        _prep_body,
        grid=(2, grid_i),
        in_specs=[
            pl.BlockSpec((BLK_MM, F), lambda p, i: (i, 0)),
            pl.BlockSpec((1, F), lambda p, i: (0, 0)),
            pl.BlockSpec((1, F), lambda p, i: (0, 0)),
            pl.BlockSpec((F, F), lambda p, i: (0, 0)),
            pl.BlockSpec((1, F), lambda p, i: (0, 0)),
        ],
        out_specs=[
            pl.BlockSpec((F, F), lambda p, i: (0, 0)),
            pl.BlockSpec((1, F), lambda p, i: (0, 0)),
            pl.BlockSpec((BLK_MM, F), lambda p, i: (p * i, 0)),
        ],
        out_shape=[
            jax.ShapeDtypeStruct((F, F), jnp.float32),
            jax.ShapeDtypeStruct((1, F), jnp.float32),
            jax.ShapeDtypeStruct((E_EDGES, F), jnp.float32),
        ],
        scratch_shapes=[
            pltpu.VMEM((1, F), jnp.float32),
            pltpu.VMEM((1, F), jnp.float32),
        ],
    )(e, g, bt, w, b)


def _sc_scatter_body(f_hbm, dst_hbm, zrow_hbm, ones_hbm, seq_hbm,
                     acc_hbm, cnt_hbm,
                     idx0, idx1, val0, val1, one_v, seq_v, acc_sh,
                     si0, si1, sv0, sv1):
    c = lax.axis_index("c")
    s = lax.axis_index("s")
    nk = ROWS_PER_TILE // WB

    pltpu.sync_copy(zrow_hbm, val0)
    pltpu.sync_copy(ones_hbm, one_v)
    pltpu.sync_copy(seq_hbm.at[pl.ds(s * nk, nk), :], seq_v)

    row0 = s * ROWS_PER_TILE
    out0 = c * N_PAD + row0
    base = (c * NS + s) * EDGES_PER_TILE

    def zero_acc():
        for k in range(nk):
            pltpu.async_copy(val0, acc_sh.at[seq_v.at[k]], sv1)
        for k in range(nk):
            pltpu.make_async_copy(val0, acc_sh.at[seq_v.at[k]], sv1).wait()

    def read_acc(dst_ref):
        bufs = (val0, val1)
        sems = (sv0, sv1)
        for k in range(nk):
            b = k % 2
            if k >= 2:
                pltpu.make_async_copy(
                    bufs[b],
                    dst_ref.at[pl.ds(out0 + (k - 2) * WB, WB), :],
                    sems[b]).wait()
            pltpu.sync_copy(acc_sh.at[seq_v.at[k]], bufs[b])
            pltpu.async_copy(bufs[b],
                             dst_ref.at[pl.ds(out0 + k * WB, WB), :], sems[b])
        for k in range(nk - 2, nk):
            b = k % 2
            pltpu.make_async_copy(
                bufs[b], dst_ref.at[pl.ds(out0 + k * WB, WB), :],
                sems[b]).wait()

    def start_fill(chunk, idx_b, val_b, s_i, s_v):
        off = base + chunk * CHUNK
        pltpu.async_copy(dst_hbm.at[pl.ds(off, CHUNK)], idx_b, s_i)
        pltpu.async_copy(f_hbm.at[pl.ds(off, CHUNK), :], val_b, s_v)

    def wait_fill(chunk, idx_b, val_b, s_i, s_v):
        off = base + chunk * CHUNK
        pltpu.make_async_copy(dst_hbm.at[pl.ds(off, CHUNK)], idx_b, s_i).wait()
        pltpu.make_async_copy(f_hbm.at[pl.ds(off, CHUNK), :], val_b, s_v).wait()

    def start_idx(chunk, idx_b, s_i):
        off = base + chunk * CHUNK
        pltpu.async_copy(dst_hbm.at[pl.ds(off, CHUNK)], idx_b, s_i)

    def wait_idx(chunk, idx_b, s_i):
        off = base + chunk * CHUNK
        pltpu.make_async_copy(dst_hbm.at[pl.ds(off, CHUNK)], idx_b, s_i).wait()

    zero_acc()
    plsc.subcore_barrier()

    start_fill(0, idx0, val0, si0, sv0)

    def pair1(p, carry):
        c0 = 2 * p
        start_fill(c0 + 1, idx1, val1, si1, sv1)
        wait_fill(c0, idx0, val0, si0, sv0)
        pltpu.sync_copy(val0, acc_sh.at[idx0], add=True)
        start_fill(c0 + 2, idx0, val0, si0, sv0)
        wait_fill(c0 + 1, idx1, val1, si1, sv1)
        pltpu.sync_copy(val1, acc_sh.at[idx1], add=True)
        return carry

    lax.fori_loop(0, (N_CHUNKS - 1) // 2, pair1, 0)
    wait_fill(N_CHUNKS - 1, idx0, val0, si0, sv0)
    pltpu.sync_copy(val0, acc_sh.at[idx0], add=True)

    plsc.subcore_barrier()
    read_acc(acc_hbm)
    plsc.subcore_barrier()

    pltpu.sync_copy(zrow_hbm, val0)
    pltpu.sync_copy(zrow_hbm, val1)
    zero_acc()
    plsc.subcore_barrier()

    start_idx(0, idx0, si0)

    def pair2(p, carry):
        c0 = 2 * p
        start_idx(c0 + 1, idx1, si1)
        wait_idx(c0, idx0, si0)
        pltpu.sync_copy(one_v, acc_sh.at[idx0], add=True)
        start_idx(c0 + 2, idx0, si0)
        wait_idx(c0 + 1, idx1, si1)
        pltpu.sync_copy(one_v, acc_sh.at[idx1], add=True)
        return carry

    lax.fori_loop(0, (N_CHUNKS - 1) // 2, pair2, 0)
    wait_idx(N_CHUNKS - 1, idx0, si0)
    pltpu.sync_copy(one_v, acc_sh.at[idx0], add=True)

    plsc.subcore_barrier()
    read_acc(cnt_hbm)


@functools.cache
def _sc_scatter_fn():
    mesh = plsc.VectorSubcoreMesh(
        core_axis_name="c", subcore_axis_name="s", num_cores=NC, num_subcores=NS
    )
    return pl.kernel(
        _sc_scatter_body,
        out_type=[
            jax.ShapeDtypeStruct((NC * N_PAD, F), jnp.float32),
            jax.ShapeDtypeStruct((NC * N_PAD, F), jnp.float32),
        ],
        mesh=mesh,
        scratch_types=[
            pltpu.VMEM((CHUNK,), jnp.int32),
            pltpu.VMEM((CHUNK,), jnp.int32),
            pltpu.VMEM((CHUNK, F), jnp.float32),
            pltpu.VMEM((CHUNK, F), jnp.float32),
            pltpu.VMEM((CHUNK, F), jnp.float32),
            pltpu.VMEM((ROWS_PER_TILE // WB, WB), jnp.int32),
            pltpu.VMEM_SHARED((N_PAD, F), jnp.float32),
            pltpu.SemaphoreType.DMA,
            pltpu.SemaphoreType.DMA,
            pltpu.SemaphoreType.DMA,
            pltpu.SemaphoreType.DMA,
        ],
    )


def _combine_body(acc_ref, cnt_ref, o_ref):
    a = acc_ref[0] + acc_ref[1]
    n = cnt_ref[0, :, 0:1] + cnt_ref[1, :, 0:1]
    o_ref[...] = a / jnp.maximum(n, 1.0)


def _combine_call(acc, cnt):
    grid = N_NODES // BLK_CMB
    return pl.pallas_call(
        _combine_body,
        grid=(grid,),
        in_specs=[
            pl.BlockSpec((NC, BLK_CMB, F), lambda i: (0, i, 0)),
            pl.BlockSpec((NC, BLK_CMB, F), lambda i: (0, i, 0)),
        ],
        out_specs=pl.BlockSpec((BLK_CMB, F), lambda i: (i, 0)),
        out_shape=jax.ShapeDtypeStruct((N_NODES, F), jnp.float32),
    )(acc, cnt)


def kernel(e, edge_index, gamma, beta, W, b):
    dst = edge_index[1].astype(jnp.int32)
    g = gamma.reshape(1, F)
    bt = beta.reshape(1, F)
    b2 = b.reshape(1, F)

    _, _, f = _prep_call(e, g, bt, W, b2)

    zrow = jnp.zeros((WB, F), jnp.float32)
    ones = jnp.ones((CHUNK, F), jnp.float32)
    seq = jnp.arange(N_PAD, dtype=jnp.int32).reshape(-1, WB)
    acc, cnt = _sc_scatter_fn()(f, dst, zrow, ones, seq)
    return _combine_call(acc.reshape(NC, N_PAD, F),
                         cnt.reshape(NC, N_PAD, F))


# speedup vs baseline: 1.1979x; 1.0134x over previous
import functools

import jax
import jax.numpy as jnp
from jax import lax
from jax.experimental import pallas as pl
from jax.experimental.pallas import tpu as pltpu
import jax.experimental.pallas.tpu_sc as plsc

N_NODES = 10000
E_EDGES = 320000
F = 128
EPS = 1e-5

NC = 2
NS = 16
N_PAD = 10240
ROWS_PER_TILE = N_PAD // NS
EDGES_PER_TILE = E_EDGES // (NC * NS)
CHUNK = 80
N_CHUNKS = EDGES_PER_TILE // CHUNK
IDX_ROWS = 128
WB = 80
CNT_W = 32

BLK_STATS = 2560
BLK_MM = 2560
BLK_CMB = 2000


def _prep_body(e_ref, g_ref, bt_ref, w_ref, b_ref, wp_ref, bp_ref, f_ref,
               s_acc, q_acc):
    p = pl.program_id(0)
    i = pl.program_id(1)

    @pl.when(p == 0)
    def _():
        blk = e_ref[...]
        s = jnp.sum(blk, axis=0, keepdims=True)
        q = jnp.sum(blk * blk, axis=0, keepdims=True)

        @pl.when(i == 0)
        def _():
            s_acc[...] = s
            q_acc[...] = q

        @pl.when(i > 0)
        def _():
            s_acc[...] += s
            q_acc[...] += q

        @pl.when(i == pl.num_programs(1) - 1)
        def _():
            inv_e = 1.0 / E_EDGES
            mean = s_acc[...] * inv_e
            var = q_acc[...] * inv_e - mean * mean
            scale = g_ref[...] * lax.rsqrt(var + EPS)
            shift = bt_ref[...] - mean * scale
            wp_ref[...] = w_ref[...] * jnp.transpose(scale)
            bp_ref[...] = (
                jnp.dot(shift, w_ref[...], preferred_element_type=jnp.float32,
                        precision=lax.Precision.HIGHEST)
                + b_ref[...]
            )

    @pl.when(p == 1)
    def _():
        x = e_ref[...]
        w = wp_ref[...]
        x_hi = x.astype(jnp.bfloat16)
        x_lo = (x - x_hi.astype(jnp.float32)).astype(jnp.bfloat16)
        w_hi = w.astype(jnp.bfloat16)
        w_lo = (w - w_hi.astype(jnp.float32)).astype(jnp.bfloat16)
        acc = jnp.dot(x_hi, w_lo, preferred_element_type=jnp.float32)
        acc = acc + jnp.dot(x_lo, w_hi, preferred_element_type=jnp.float32)
        acc = acc + jnp.dot(x_hi, w_hi, preferred_element_type=jnp.float32)
        f_ref[...] = jnp.maximum(acc + bp_ref[...], 0.0)


def _prep_call(e, g, bt, w, b):
    grid_i = E_EDGES // BLK_MM
    return pl.pallas_call(
        _prep_body,
        grid=(2, grid_i),
        in_specs=[
            pl.BlockSpec((BLK_MM, F), lambda p, i: (i, 0)),
            pl.BlockSpec((1, F), lambda p, i: (0, 0)),
            pl.BlockSpec((1, F), lambda p, i: (0, 0)),
            pl.BlockSpec((F, F), lambda p, i: (0, 0)),
            pl.BlockSpec((1, F), lambda p, i: (0, 0)),
        ],
        out_specs=[
            pl.BlockSpec((F, F), lambda p, i: (0, 0)),
            pl.BlockSpec((1, F), lambda p, i: (0, 0)),
            pl.BlockSpec((BLK_MM, F), lambda p, i: (p * i, 0)),
        ],
        out_shape=[
            jax.ShapeDtypeStruct((F, F), jnp.float32),
            jax.ShapeDtypeStruct((1, F), jnp.float32),
            jax.ShapeDtypeStruct((E_EDGES, F), jnp.float32),
        ],
        scratch_shapes=[
            pltpu.VMEM((1, F), jnp.float32),
            pltpu.VMEM((1, F), jnp.float32),
        ],
    )(e, g, bt, w, b)


def _sc_scatter_body(f_hbm, dst_hbm, zrow_hbm, ones_hbm, seq_hbm,
                     acc_hbm, cnt_hbm,
                     idx_all, val0, val1, seq_v, acc_sh,
                     si0, si1, sv0, sv1):
    c = lax.axis_index("c")
    s = lax.axis_index("s")
    nk = ROWS_PER_TILE // WB

    pltpu.sync_copy(zrow_hbm, val0)
    pltpu.sync_copy(seq_hbm.at[pl.ds(s * nk, nk), :], seq_v)

    row0 = s * ROWS_PER_TILE
    out0 = c * N_PAD + row0
    wid = c * NS + s
    base = wid * EDGES_PER_TILE
    pltpu.sync_copy(dst_hbm.at[pl.ds(wid * IDX_ROWS, IDX_ROWS), :], idx_all)

    def zero_acc():
        for k in range(nk):
            pltpu.async_copy(val0, acc_sh.at[seq_v.at[k]], sv1)
        for k in range(nk):
            pltpu.make_async_copy(val0, acc_sh.at[seq_v.at[k]], sv1).wait()

    def read_acc(dst_ref):
        bufs = (val0, val1)
        sems = (sv0, sv1)
        for k in range(nk):
            b = k % 2
            if k >= 2:
                pltpu.make_async_copy(
                    bufs[b],
                    dst_ref.at[pl.ds(out0 + (k - 2) * WB, WB), :],
                    sems[b]).wait()
            pltpu.sync_copy(acc_sh.at[seq_v.at[k]], bufs[b])
            pltpu.async_copy(bufs[b],
                             dst_ref.at[pl.ds(out0 + k * WB, WB), :], sems[b])
        for k in range(nk - 2, nk):
            b = k % 2
            pltpu.make_async_copy(
                bufs[b], dst_ref.at[pl.ds(out0 + k * WB, WB), :],
                sems[b]).wait()

    def start_fill(chunk, val_b, s_v):
        off = base + chunk * CHUNK
        pltpu.async_copy(f_hbm.at[pl.ds(off, CHUNK), :], val_b, s_v)

    def wait_fill(chunk, val_b, s_v):
        off = base + chunk * CHUNK
        pltpu.make_async_copy(f_hbm.at[pl.ds(off, CHUNK), :], val_b, s_v).wait()

    zero_acc()
    plsc.subcore_barrier()

    start_fill(0, val0, sv0)

    def pair1(p, carry):
        c0 = 2 * p
        start_fill(c0 + 1, val1, sv1)
        wait_fill(c0, val0, sv0)
        pltpu.sync_copy(val0, acc_sh.at[idx_all.at[c0]], add=True)
        start_fill(c0 + 2, val0, sv0)
        wait_fill(c0 + 1, val1, sv1)
        pltpu.sync_copy(val1, acc_sh.at[idx_all.at[c0 + 1]], add=True)
        return carry

    lax.fori_loop(0, (N_CHUNKS - 1) // 2, pair1, 0)
    wait_fill(N_CHUNKS - 1, val0, sv0)
    pltpu.sync_copy(val0, acc_sh.at[idx_all.at[N_CHUNKS - 1]], add=True)

    plsc.subcore_barrier()
    read_acc(acc_hbm)
    plsc.subcore_barrier()

    pltpu.sync_copy(zrow_hbm, val0)
    pltpu.sync_copy(ones_hbm, val1)
    zero_acc()
    plsc.subcore_barrier()

    def grp2(p, carry):
        c0 = p * 8
        for k in range(8):
            pltpu.async_copy(val1, acc_sh.at[idx_all.at[c0 + k]], si0)
        for k in range(8):
            pltpu.make_async_copy(val1, acc_sh.at[idx_all.at[c0 + k]],
                                  si0).wait()
        return carry

    lax.fori_loop(0, N_CHUNKS // 8, grp2, 0)
    for k in range(N_CHUNKS - (N_CHUNKS // 8) * 8):
        c0 = (N_CHUNKS // 8) * 8 + k
        pltpu.async_copy(val1, acc_sh.at[idx_all.at[c0]], si1)
        pltpu.make_async_copy(val1, acc_sh.at[idx_all.at[c0]], si1).wait()

    plsc.subcore_barrier()
    read_acc(cnt_hbm)


@functools.cache
def _sc_scatter_fn():
    mesh = plsc.VectorSubcoreMesh(
        core_axis_name="c", subcore_axis_name="s", num_cores=NC, num_subcores=NS
    )
    return pl.kernel(
        _sc_scatter_body,
        out_type=[
            jax.ShapeDtypeStruct((NC * N_PAD, F), jnp.float32),
            jax.ShapeDtypeStruct((NC * N_PAD, F), jnp.float32),
        ],
        mesh=mesh,
        scratch_types=[
            pltpu.VMEM((IDX_ROWS, CHUNK), jnp.int32),
            pltpu.VMEM((CHUNK, F), jnp.float32),
            pltpu.VMEM((CHUNK, F), jnp.float32),
            pltpu.VMEM((ROWS_PER_TILE // WB, WB), jnp.int32),
            pltpu.VMEM_SHARED((N_PAD, F), jnp.float32),
            pltpu.SemaphoreType.DMA,
            pltpu.SemaphoreType.DMA,
            pltpu.SemaphoreType.DMA,
            pltpu.SemaphoreType.DMA,
        ],
    )


def _combine_body(acc_ref, cnt_ref, o_ref):
    a = acc_ref[0] + acc_ref[1]
    n = cnt_ref[0, :, 0:1] + cnt_ref[1, :, 0:1]
    o_ref[...] = a / jnp.maximum(n, 1.0)


def _combine_call(acc, cnt):
    grid = N_NODES // BLK_CMB
    return pl.pallas_call(
        _combine_body,
        grid=(grid,),
        in_specs=[
            pl.BlockSpec((NC, BLK_CMB, F), lambda i: (0, i, 0)),
            pl.BlockSpec((NC, BLK_CMB, F), lambda i: (0, i, 0)),
        ],
        out_specs=pl.BlockSpec((BLK_CMB, F), lambda i: (i, 0)),
        out_shape=jax.ShapeDtypeStruct((N_NODES, F), jnp.float32),
    )(acc, cnt)


def kernel(e, edge_index, gamma, beta, W, b):
    dst_flat = edge_index[1].astype(jnp.int32)
    dst = jnp.pad(dst_flat.reshape(NC * NS, N_CHUNKS, CHUNK),
                  ((0, 0), (0, IDX_ROWS - N_CHUNKS), (0, 0))
                  ).reshape(NC * NS * IDX_ROWS, CHUNK)
    g = gamma.reshape(1, F)
    bt = beta.reshape(1, F)
    b2 = b.reshape(1, F)

    _, _, f = _prep_call(e, g, bt, W, b2)

    zrow = jnp.zeros((WB, F), jnp.float32)
    ones = jnp.ones((CHUNK, F), jnp.float32)
    seq = jnp.arange(N_PAD, dtype=jnp.int32).reshape(-1, WB)
    acc, cnt = _sc_scatter_fn()(f, dst, zrow, ones, seq)
    return _combine_call(acc.reshape(NC, N_PAD, F),
                         cnt.reshape(NC, N_PAD, F))
